# baseline (speedup 1.0000x reference)
import jax
import jax.numpy as jnp
from jax.experimental import pallas as pl

_FREQ_MASK_PARAM = 27
_TIME_MASK_PARAM = 100
_N_FREQ_MASKS = 2
_N_TIME_MASKS = 2
_TIME_MASK_RATIO = 0.05

_TBLK = 600


def _mask_body(x_ref, tk_ref, fk_ref, o_ref):
    o_ref[...] = x_ref[...] * tk_ref[...] * fk_ref[...]


def _mask_bounds(B, T, F):
    key = jax.random.key(42)
    fbounds = []
    for _ in range(_N_FREQ_MASKS):
        key, k1, k2 = jax.random.split(key, 3)
        f = jax.random.randint(k1, (B,), 0, _FREQ_MASK_PARAM + 1)
        f0 = jax.random.randint(k2, (B,), 0, max(1, F - _FREQ_MASK_PARAM))
        fbounds.append((f0, f0 + f))
    t_max = _TIME_MASK_PARAM
    if _TIME_MASK_RATIO is not None:
        t_max = min(t_max, int(_TIME_MASK_RATIO * T))
    tbounds = []
    for _ in range(_N_TIME_MASKS):
        key, k1, k2 = jax.random.split(key, 3)
        t = jax.random.randint(k1, (B,), 0, max(1, t_max + 1))
        t0 = jax.random.randint(k2, (B,), 0, max(1, T - t_max))
        tbounds.append((t0, t0 + t))
    return fbounds, tbounds


def kernel(x):
    B, T, F = x.shape
    fbounds, tbounds = _mask_bounds(B, T, F)

    col = jnp.arange(F, dtype=jnp.int32)[None, :]
    fkeep = jnp.ones((B, F), dtype=x.dtype)
    for lo, hi in fbounds:
        fkeep = fkeep * ((col < lo[:, None]) | (col >= hi[:, None])).astype(x.dtype)
    row = jnp.arange(T, dtype=jnp.int32)[None, :]
    tkeep = jnp.ones((B, T), dtype=x.dtype)
    for lo, hi in tbounds:
        tkeep = tkeep * ((row < lo[:, None]) | (row >= hi[:, None])).astype(x.dtype)

    tkeep = tkeep[:, :, None]
    fkeep = fkeep[:, None, :]

    nt = T // _TBLK
    return pl.pallas_call(
        _mask_body,
        grid=(B, nt),
        in_specs=[
            pl.BlockSpec((1, _TBLK, F), lambda b, t: (b, t, 0)),
            pl.BlockSpec((1, _TBLK, 1), lambda b, t: (b, t, 0)),
            pl.BlockSpec((1, 1, F), lambda b, t: (b, 0, 0)),
        ],
        out_specs=pl.BlockSpec((1, _TBLK, F), lambda b, t: (b, t, 0)),
        out_shape=jax.ShapeDtypeStruct((B, T, F), x.dtype),
    )(x, tkeep, fkeep)


# speedup vs baseline: 1.2620x; 1.2620x over previous
import jax
import jax.numpy as jnp
from jax.experimental import pallas as pl
from jax.experimental.pallas import tpu as pltpu

_FREQ_MASK_PARAM = 27
_TIME_MASK_PARAM = 100
_N_FREQ_MASKS = 2
_N_TIME_MASKS = 2
_TIME_MASK_RATIO = 0.05

_TBLK = 1000


def _mask_body(s_ref, x_ref, o_ref):
    b = pl.program_id(0)
    base = pl.program_id(1) * _TBLK
    _, tblk, F = x_ref.shape
    rows = jax.lax.broadcasted_iota(jnp.int32, (1, tblk, 1), 1) + base
    cols = jax.lax.broadcasted_iota(jnp.int32, (1, 1, F), 2)
    tkeep = ((rows < s_ref[4, b]) | (rows >= s_ref[5, b])) & (
        (rows < s_ref[6, b]) | (rows >= s_ref[7, b]))
    fkeep = ((cols < s_ref[0, b]) | (cols >= s_ref[1, b])) & (
        (cols < s_ref[2, b]) | (cols >= s_ref[3, b]))
    o_ref[...] = jnp.where(tkeep & fkeep, x_ref[...], jnp.float32(0.0))


def _mask_bounds(B, T, F):
    key = jax.random.key(42)
    rows = []
    for _ in range(_N_FREQ_MASKS):
        key, k1, k2 = jax.random.split(key, 3)
        f = jax.random.randint(k1, (B,), 0, _FREQ_MASK_PARAM + 1)
        f0 = jax.random.randint(k2, (B,), 0, max(1, F - _FREQ_MASK_PARAM))
        rows += [f0, f0 + f]
    t_max = _TIME_MASK_PARAM
    if _TIME_MASK_RATIO is not None:
        t_max = min(t_max, int(_TIME_MASK_RATIO * T))
    for _ in range(_N_TIME_MASKS):
        key, k1, k2 = jax.random.split(key, 3)
        t = jax.random.randint(k1, (B,), 0, max(1, t_max + 1))
        t0 = jax.random.randint(k2, (B,), 0, max(1, T - t_max))
        rows += [t0, t0 + t]
    return jnp.stack(rows).astype(jnp.int32)


def kernel(x):
    B, T, F = x.shape
    bounds = _mask_bounds(B, T, F)
    return pl.pallas_call(
        _mask_body,
        grid=(B, T // _TBLK),
        in_specs=[
            pl.BlockSpec(memory_space=pltpu.SMEM),
            pl.BlockSpec((1, _TBLK, F), lambda b, t: (b, t, 0)),
        ],
        out_specs=pl.BlockSpec((1, _TBLK, F), lambda b, t: (b, t, 0)),
        out_shape=jax.ShapeDtypeStruct((B, T, F), x.dtype),
    )(bounds, x)
